# baseline (speedup 1.0000x reference)
import functools

import numpy as np
import jax
import jax.numpy as jnp
from jax import lax
from jax.experimental import pallas as pl
from jax.experimental.pallas import tpu as pltpu
from jax.experimental.pallas import tpu_sc as plsc

PAD = 0
V = 1000000
D = 32
L = 16
IB = 128

_INFO = plsc.get_sparse_core_info()
NC, NS = _INFO.num_cores, _INFO.num_subcores
NW = NC * NS

NFULL = V // 128
TAIL = V - NFULL * 128
BASE_BLKS = NFULL // NW
EXTRA = NFULL - BASE_BLKS * NW


def _format_table(weight_t):
    mesh = plsc.VectorSubcoreMesh(core_axis_name="c", subcore_axis_name="s")

    @functools.partial(
        pl.kernel,
        mesh=mesh,
        out_type=jax.ShapeDtypeStruct((V // 4, 128), jnp.float32),
        compiler_params=pltpu.CompilerParams(
            needs_layout_passes=False, use_tc_tiling_on_sc=True
        ),
        scratch_types=[
            pltpu.VMEM((D, 128), jnp.float32),
            pltpu.VMEM((D, 128), jnp.float32),
        ],
    )
    def k(wt_hbm, out_hbm, s_v, o_v):
        wid = lax.axis_index("s") * NC + lax.axis_index("c")
        nblk = jnp.where(wid < EXTRA, BASE_BLKS + 1, BASE_BLKS)
        b0 = wid * BASE_BLKS + jnp.minimum(wid, EXTRA)

        def transpose_block(ncols):
            for k16 in range(8):
                d0 = 16 * (k16 % 2)
                sub = k16 // 2
                rows = lax.iota(jnp.int32, L) + d0
                for q in range(ncols // 4):
                    col = jnp.full((L,), 4 * q + sub, jnp.int32)
                    o_v[q, pl.ds(16 * k16, L)] = plsc.load_gather(
                        s_v, [rows, col]
                    )

        def blk_body(i, _):
            b = b0 + i
            c0 = pl.multiple_of(b * 128, 128)
            pltpu.sync_copy(wt_hbm.at[:, pl.ds(c0, 128)], s_v)
            transpose_block(128)
            pltpu.sync_copy(o_v, out_hbm.at[pl.ds(pl.multiple_of(b * 32, 32), 32)])
            return 0

        lax.fori_loop(0, nblk, blk_body, 0, unroll=False)

        @pl.when(wid == NW - 1)
        def _():
            for d in range(D):
                pltpu.sync_copy(
                    wt_hbm.at[d, pl.ds(NFULL * 128, TAIL)],
                    s_v.at[d, pl.ds(0, TAIL)],
                )
            for k16 in range(8):
                d0 = 16 * (k16 % 2)
                sub = k16 // 2
                rows = lax.iota(jnp.int32, L) + d0
                for q in range(TAIL // 4):
                    col = jnp.full((L,), 4 * q + sub, jnp.int32)
                    o_v[q, pl.ds(16 * k16, L)] = plsc.load_gather(
                        s_v, [rows, col]
                    )
            pltpu.sync_copy(
                o_v.at[pl.ds(0, TAIL // 4)],
                out_hbm.at[pl.ds(NFULL * 32, TAIL // 4)],
            )

    return k(weight_t)


def _lane_min(v):
    dnums = lax.GatherDimensionNumbers(
        offset_dims=(), collapsed_slice_dims=(0,), start_index_map=(0,)
    )
    for sh in (8, 4, 2, 1):
        perm = (lax.iota(jnp.int32, L) + sh) % L
        rot = lax.gather(
            v, perm[:, None], dnums, (1,),
            mode=lax.GatherScatterMode.PROMISE_IN_BOUNDS,
        )
        v = jnp.minimum(v, rot)
    return v[0]


def _gather_rows(table, idx, B, C):
    b_per_w = B // NW
    n_chunks = b_per_w // C
    gathers_per_chunk = C // IB
    mesh = plsc.VectorSubcoreMesh(core_axis_name="c", subcore_axis_name="s")

    @functools.partial(
        pl.kernel,
        mesh=mesh,
        out_type=jax.ShapeDtypeStruct((B, D), jnp.float32),
        compiler_params=pltpu.CompilerParams(
            needs_layout_passes=False, use_tc_tiling_on_sc=False
        ),
        scratch_types=[
            pltpu.VMEM((C,), jnp.int32),
            pltpu.VMEM((C, D), jnp.float32),
            pltpu.SemaphoreType.DMA,
        ],
    )
    def k(table_hbm, idx_hbm, out_hbm, idx_v, rows_v, sem):
        wid = lax.axis_index("s") * NC + lax.axis_index("c")
        base = wid * b_per_w
        zeros = jnp.zeros((L,), jnp.float32)

        def chunk_body(g, _):
            off = pl.multiple_of(base + g * C, C)
            pltpu.sync_copy(idx_hbm.at[pl.ds(off, C)], idx_v)
            for j in range(gathers_per_chunk):
                pltpu.async_copy(
                    table_hbm.at[idx_v.at[pl.ds(j * IB, IB)]],
                    rows_v.at[pl.ds(j * IB, IB)],
                    sem,
                )

            def min_body(i, acc):
                return jnp.minimum(acc, idx_v[pl.ds(i * L, L)])

            acc = lax.fori_loop(
                0, C // L, min_body, jnp.full((L,), 2**30, jnp.int32),
                unroll=False,
            )
            min_idx = _lane_min(acc)

            for j in range(gathers_per_chunk):
                pltpu.make_async_copy(
                    table_hbm.at[idx_v.at[pl.ds(j * IB, IB)]],
                    rows_v.at[pl.ds(j * IB, IB)],
                    sem,
                ).wait()

            @pl.when(min_idx == PAD)
            def _():
                def fix_body(i, _):
                    iv = idx_v[pl.ds(i * L, L)]
                    m = iv == PAD
                    rowpos = lax.iota(jnp.int32, L) + i * L
                    for j in range(D):
                        plsc.store_scatter(
                            rows_v,
                            [rowpos, jnp.full((L,), j, jnp.int32)],
                            zeros,
                            mask=m,
                        )
                    return 0

                lax.fori_loop(0, C // L, fix_body, 0, unroll=False)

            pltpu.sync_copy(rows_v, out_hbm.at[pl.ds(off, C)])
            return 0

        lax.fori_loop(0, n_chunks, chunk_body, 0, unroll=False)

    return k(table, idx)


@functools.partial(jax.jit, static_argnums=(2,))
def _emb_lookup(weight, idx, B):
    table_packed = _format_table(jnp.swapaxes(weight, 0, 1))
    table = table_packed.reshape(V, D)
    return _gather_rows(table, idx, B, 1024)


def kernel(input_, weight):
    shape = input_.shape
    B = input_.size
    idx = input_.reshape(B)
    out = _emb_lookup(weight, idx, B)
    return out.reshape(*shape, D)


# speedup vs baseline: 1.0442x; 1.0442x over previous
import functools

import numpy as np
import jax
import jax.numpy as jnp
from jax import lax
from jax.experimental import pallas as pl
from jax.experimental.pallas import tpu as pltpu
from jax.experimental.pallas import tpu_sc as plsc

PAD = 0
V = 1000000
D = 32
L = 16
IB = 128

_INFO = plsc.get_sparse_core_info()
NC, NS = _INFO.num_cores, _INFO.num_subcores
NW = NC * NS

NFULL = V // 128
TAIL = V - NFULL * 128
BASE_BLKS = NFULL // NW
EXTRA = NFULL - BASE_BLKS * NW


SP = 129
NBUF = 4


def _format_table(weight_t):
    mesh = plsc.VectorSubcoreMesh(core_axis_name="c", subcore_axis_name="s")

    @functools.partial(
        pl.kernel,
        mesh=mesh,
        out_type=jax.ShapeDtypeStruct((V // 4, 128), jnp.float32),
        compiler_params=pltpu.CompilerParams(
            needs_layout_passes=False, use_tc_tiling_on_sc=True
        ),
        scratch_types=[pltpu.VMEM((NBUF, D, SP), jnp.float32)]
        + [pltpu.VMEM((NBUF, D, 128), jnp.float32)]
        + [pltpu.SemaphoreType.DMA] * NBUF
        + [pltpu.SemaphoreType.DMA] * NBUF,
    )
    def k(wt_hbm, out_hbm, s_v, o_v, *sems):
        sin = sems[:NBUF]
        sout = sems[NBUF:]
        wid = lax.axis_index("s") * NC + lax.axis_index("c")
        nblk = jnp.where(wid < EXTRA, BASE_BLKS + 1, BASE_BLKS)
        b0 = wid * BASE_BLKS + jnp.minimum(wid, EXTRA)

        def start_in(g, buf):
            c0 = pl.multiple_of((b0 + g) * 128, 128)
            pltpu.async_copy(
                wt_hbm.at[:, pl.ds(c0, 128)],
                s_v.at[buf, :, pl.ds(0, 128)],
                sin[buf],
            )

        def wait_in(g, buf):
            c0 = pl.multiple_of((b0 + g) * 128, 128)
            pltpu.make_async_copy(
                wt_hbm.at[:, pl.ds(c0, 128)],
                s_v.at[buf, :, pl.ds(0, 128)],
                sin[buf],
            ).wait()

        def start_out(g, buf):
            p0 = pl.multiple_of((b0 + g) * 32, 32)
            pltpu.async_copy(o_v.at[buf], out_hbm.at[pl.ds(p0, 32)], sout[buf])

        def wait_out(g, buf):
            p0 = pl.multiple_of((b0 + g) * 32, 32)
            pltpu.make_async_copy(
                o_v.at[buf], out_hbm.at[pl.ds(p0, 32)], sout[buf]
            ).wait()

        def transpose_block(buf, ncols):
            for k16 in range(8):
                rows = lax.iota(jnp.int32, L) + 16 * (k16 % 2)
                sub = k16 // 2
                for q in range(ncols // 4):
                    col = jnp.full((L,), 4 * q + sub, jnp.int32)
                    o_v[buf, q, pl.ds(16 * k16, L)] = plsc.load_gather(
                        s_v.at[buf], [rows, col]
                    )

        for buf in range(NBUF):
            @pl.when(buf < nblk)
            def _(buf=buf):
                start_in(buf, buf)

        def ring_body(i, _):
            for buf in range(NBUF):
                g = i * NBUF + buf

                @pl.when(g < nblk)
                def _(g=g, buf=buf):
                    wait_in(g, buf)

                    @pl.when(g >= NBUF)
                    def _():
                        wait_out(g - NBUF, buf)

                    transpose_block(buf, 128)
                    start_out(g, buf)

                    @pl.when(g + NBUF < nblk)
                    def _():
                        start_in(g + NBUF, buf)

            return 0

        n_iters = (BASE_BLKS + 1 + NBUF - 1) // NBUF
        lax.fori_loop(0, n_iters, ring_body, 0, unroll=False)

        for buf in range(NBUF):
            wait_out(0, buf)

        @pl.when(wid == NW - 1)
        def _():
            for d in range(D):
                pltpu.sync_copy(
                    wt_hbm.at[d, pl.ds(NFULL * 128, TAIL)],
                    s_v.at[0, d, pl.ds(0, TAIL)],
                )
            for k16 in range(8):
                rows = lax.iota(jnp.int32, L) + 16 * (k16 % 2)
                sub = k16 // 2
                for q in range(TAIL // 4):
                    col = jnp.full((L,), 4 * q + sub, jnp.int32)
                    o_v[0, q, pl.ds(16 * k16, L)] = plsc.load_gather(
                        s_v.at[0], [rows, col]
                    )
            pltpu.sync_copy(
                o_v.at[0, pl.ds(0, TAIL // 4)],
                out_hbm.at[pl.ds(NFULL * 32, TAIL // 4)],
            )

    return k(weight_t)


def _lane_min(v):
    dnums = lax.GatherDimensionNumbers(
        offset_dims=(), collapsed_slice_dims=(0,), start_index_map=(0,)
    )
    for sh in (8, 4, 2, 1):
        perm = (lax.iota(jnp.int32, L) + sh) % L
        rot = lax.gather(
            v, perm[:, None], dnums, (1,),
            mode=lax.GatherScatterMode.PROMISE_IN_BOUNDS,
        )
        v = jnp.minimum(v, rot)
    return v[0]


def _gather_rows(table, idx, B, C):
    b_per_w = B // NW
    n_chunks = b_per_w // C
    gathers_per_chunk = C // IB
    mesh = plsc.VectorSubcoreMesh(core_axis_name="c", subcore_axis_name="s")

    @functools.partial(
        pl.kernel,
        mesh=mesh,
        out_type=jax.ShapeDtypeStruct((B, D), jnp.float32),
        compiler_params=pltpu.CompilerParams(
            needs_layout_passes=False, use_tc_tiling_on_sc=False
        ),
        scratch_types=[
            pltpu.VMEM((C,), jnp.int32),
            pltpu.VMEM((C, D), jnp.float32),
            pltpu.SemaphoreType.DMA,
        ],
    )
    def k(table_hbm, idx_hbm, out_hbm, idx_v, rows_v, sem):
        wid = lax.axis_index("s") * NC + lax.axis_index("c")
        base = wid * b_per_w
        zeros = jnp.zeros((L,), jnp.float32)

        def chunk_body(g, _):
            off = pl.multiple_of(base + g * C, C)
            pltpu.sync_copy(idx_hbm.at[pl.ds(off, C)], idx_v)
            for j in range(gathers_per_chunk):
                pltpu.async_copy(
                    table_hbm.at[idx_v.at[pl.ds(j * IB, IB)]],
                    rows_v.at[pl.ds(j * IB, IB)],
                    sem,
                )

            def min_body(i, acc):
                return jnp.minimum(acc, idx_v[pl.ds(i * L, L)])

            acc = lax.fori_loop(
                0, C // L, min_body, jnp.full((L,), 2**30, jnp.int32),
                unroll=False,
            )
            min_idx = _lane_min(acc)

            for j in range(gathers_per_chunk):
                pltpu.make_async_copy(
                    table_hbm.at[idx_v.at[pl.ds(j * IB, IB)]],
                    rows_v.at[pl.ds(j * IB, IB)],
                    sem,
                ).wait()

            @pl.when(min_idx == PAD)
            def _():
                def fix_body(i, _):
                    iv = idx_v[pl.ds(i * L, L)]
                    m = iv == PAD
                    rowpos = lax.iota(jnp.int32, L) + i * L
                    for j in range(D):
                        plsc.store_scatter(
                            rows_v,
                            [rowpos, jnp.full((L,), j, jnp.int32)],
                            zeros,
                            mask=m,
                        )
                    return 0

                lax.fori_loop(0, C // L, fix_body, 0, unroll=False)

            pltpu.sync_copy(rows_v, out_hbm.at[pl.ds(off, C)])
            return 0

        lax.fori_loop(0, n_chunks, chunk_body, 0, unroll=False)

    return k(table, idx)


@functools.partial(jax.jit, static_argnums=(2,))
def _emb_lookup(weight, idx, B):
    table_packed = _format_table(jnp.swapaxes(weight, 0, 1))
    table = table_packed.reshape(V, D)
    return _gather_rows(table, idx, B, 1024)


def kernel(input_, weight):
    shape = input_.shape
    B = input_.size
    idx = input_.reshape(B)
    out = _emb_lookup(weight, idx, B)
    return out.reshape(*shape, D)


# speedup vs baseline: 1.3560x; 1.2986x over previous
import functools

import numpy as np
import jax
import jax.numpy as jnp
from jax import lax
from jax.experimental import pallas as pl
from jax.experimental.pallas import tpu as pltpu
from jax.experimental.pallas import tpu_sc as plsc

PAD = 0
V = 1000000
D = 32
L = 16
IB = 128

_INFO = plsc.get_sparse_core_info()
NC, NS = _INFO.num_cores, _INFO.num_subcores
NW = NC * NS

NFULL = V // 128
TAIL = V - NFULL * 128
BASE_BLKS = NFULL // NW
EXTRA = NFULL - BASE_BLKS * NW


WO = 128
NBUF = 4


def _format_table(weight_t):
    mesh = plsc.VectorSubcoreMesh(core_axis_name="c", subcore_axis_name="s")

    @functools.partial(
        pl.kernel,
        mesh=mesh,
        out_type=jax.ShapeDtypeStruct((V // 4, 128), jnp.float32),
        compiler_params=pltpu.CompilerParams(
            needs_layout_passes=False, use_tc_tiling_on_sc=True
        ),
        scratch_types=[pltpu.VMEM((NBUF, D, 128), jnp.float32)]
        + [pltpu.VMEM((NBUF, D, WO), jnp.float32)]
        + [pltpu.SemaphoreType.DMA] * NBUF
        + [pltpu.SemaphoreType.DMA] * NBUF,
    )
    def k(wt_hbm, out_hbm, s_v, o_v, *sems):
        sin = sems[:NBUF]
        sout = sems[NBUF:]
        wid = lax.axis_index("s") * NC + lax.axis_index("c")
        nblk = jnp.where(wid < EXTRA, BASE_BLKS + 1, BASE_BLKS)
        b0 = wid * BASE_BLKS + jnp.minimum(wid, EXTRA)

        lanes = lax.iota(jnp.int32, L)
        rows_g = [(lanes + 16 * g) // 4 for g in range(8)]
        cols_g = [((lanes + 16 * g) % 4) * 32 for g in range(8)]

        def start_in(g, buf):
            c0 = pl.multiple_of((b0 + g) * 128, 128)
            pltpu.async_copy(
                wt_hbm.at[:, pl.ds(c0, 128)], s_v.at[buf], sin[buf]
            )

        def wait_in(g, buf):
            c0 = pl.multiple_of((b0 + g) * 128, 128)
            pltpu.make_async_copy(
                wt_hbm.at[:, pl.ds(c0, 128)], s_v.at[buf], sin[buf]
            ).wait()

        def start_out(g, buf):
            p0 = pl.multiple_of((b0 + g) * 32, 32)
            pltpu.async_copy(
                o_v.at[buf], out_hbm.at[pl.ds(p0, 32)], sout[buf]
            )

        def wait_out(g, buf):
            p0 = pl.multiple_of((b0 + g) * 32, 32)
            pltpu.make_async_copy(
                o_v.at[buf], out_hbm.at[pl.ds(p0, 32)], sout[buf]
            ).wait()

        def transpose_block(buf, ncols):
            for d in range(D):
                for g in range(ncols // 16):
                    vals = s_v[buf, d, pl.ds(16 * g, L)]
                    plsc.store_scatter(
                        o_v.at[buf], [rows_g[g], cols_g[g] + d], vals
                    )

        for buf in range(NBUF):
            @pl.when(buf < nblk)
            def _(buf=buf):
                start_in(buf, buf)

        def ring_body(i, _):
            for buf in range(NBUF):
                g = i * NBUF + buf

                @pl.when(g < nblk)
                def _(g=g, buf=buf):
                    wait_in(g, buf)

                    @pl.when(g >= NBUF)
                    def _():
                        wait_out(g - NBUF, buf)

                    transpose_block(buf, 128)
                    start_out(g, buf)

                    @pl.when(g + NBUF < nblk)
                    def _():
                        start_in(g + NBUF, buf)

            return 0

        n_iters = (BASE_BLKS + 1 + NBUF - 1) // NBUF
        lax.fori_loop(0, n_iters, ring_body, 0, unroll=False)

        for buf in range(NBUF):
            wait_out(0, buf)

        @pl.when(wid == NW - 1)
        def _():
            for d in range(D):
                pltpu.sync_copy(
                    wt_hbm.at[d, pl.ds(NFULL * 128, TAIL)],
                    s_v.at[0, d, pl.ds(0, TAIL)],
                )
            for d in range(D):
                for g in range(TAIL // 16):
                    vals = s_v[0, d, pl.ds(16 * g, L)]
                    plsc.store_scatter(
                        o_v.at[0], [rows_g[g], cols_g[g] + d], vals
                    )
            pltpu.sync_copy(
                o_v.at[0, pl.ds(0, TAIL // 4)],
                out_hbm.at[pl.ds(NFULL * 32, TAIL // 4)],
            )

    return k(weight_t)


def _lane_min(v):
    dnums = lax.GatherDimensionNumbers(
        offset_dims=(), collapsed_slice_dims=(0,), start_index_map=(0,)
    )
    for sh in (8, 4, 2, 1):
        perm = (lax.iota(jnp.int32, L) + sh) % L
        rot = lax.gather(
            v, perm[:, None], dnums, (1,),
            mode=lax.GatherScatterMode.PROMISE_IN_BOUNDS,
        )
        v = jnp.minimum(v, rot)
    return v[0]


def _gather_rows(table, idx, B, C):
    b_per_w = B // NW
    n_chunks = b_per_w // C
    gathers_per_chunk = C // IB
    mesh = plsc.VectorSubcoreMesh(core_axis_name="c", subcore_axis_name="s")

    @functools.partial(
        pl.kernel,
        mesh=mesh,
        out_type=jax.ShapeDtypeStruct((B, D), jnp.float32),
        compiler_params=pltpu.CompilerParams(
            needs_layout_passes=False, use_tc_tiling_on_sc=False
        ),
        scratch_types=[
            pltpu.VMEM((C,), jnp.int32),
            pltpu.VMEM((C, D), jnp.float32),
            pltpu.SemaphoreType.DMA,
        ],
    )
    def k(table_hbm, idx_hbm, out_hbm, idx_v, rows_v, sem):
        wid = lax.axis_index("s") * NC + lax.axis_index("c")
        base = wid * b_per_w
        zeros = jnp.zeros((L,), jnp.float32)

        def chunk_body(g, _):
            off = pl.multiple_of(base + g * C, C)
            pltpu.sync_copy(idx_hbm.at[pl.ds(off, C)], idx_v)
            for j in range(gathers_per_chunk):
                pltpu.async_copy(
                    table_hbm.at[idx_v.at[pl.ds(j * IB, IB)]],
                    rows_v.at[pl.ds(j * IB, IB)],
                    sem,
                )

            def min_body(i, acc):
                return jnp.minimum(acc, idx_v[pl.ds(i * L, L)])

            acc = lax.fori_loop(
                0, C // L, min_body, jnp.full((L,), 2**30, jnp.int32),
                unroll=False,
            )
            min_idx = _lane_min(acc)

            for j in range(gathers_per_chunk):
                pltpu.make_async_copy(
                    table_hbm.at[idx_v.at[pl.ds(j * IB, IB)]],
                    rows_v.at[pl.ds(j * IB, IB)],
                    sem,
                ).wait()

            @pl.when(min_idx == PAD)
            def _():
                def fix_body(i, _):
                    iv = idx_v[pl.ds(i * L, L)]
                    m = iv == PAD
                    rowpos = lax.iota(jnp.int32, L) + i * L
                    for j in range(D):
                        plsc.store_scatter(
                            rows_v,
                            [rowpos, jnp.full((L,), j, jnp.int32)],
                            zeros,
                            mask=m,
                        )
                    return 0

                lax.fori_loop(0, C // L, fix_body, 0, unroll=False)

            pltpu.sync_copy(rows_v, out_hbm.at[pl.ds(off, C)])
            return 0

        lax.fori_loop(0, n_chunks, chunk_body, 0, unroll=False)

    return k(table, idx)


@functools.partial(jax.jit, static_argnums=(2,))
def _emb_lookup(weight, idx, B):
    table_packed = _format_table(jnp.swapaxes(weight, 0, 1))
    table = table_packed.reshape(V, D)
    return _gather_rows(table, idx, B, 1024)


def kernel(input_, weight):
    shape = input_.shape
    B = input_.size
    idx = input_.reshape(B)
    out = _emb_lookup(weight, idx, B)
    return out.reshape(*shape, D)


# speedup vs baseline: 1.3668x; 1.0080x over previous
import functools

import numpy as np
import jax
import jax.numpy as jnp
from jax import lax
from jax.experimental import pallas as pl
from jax.experimental.pallas import tpu as pltpu
from jax.experimental.pallas import tpu_sc as plsc

PAD = 0
V = 1000000
D = 32
L = 16
IB = 128

_INFO = plsc.get_sparse_core_info()
NC, NS = _INFO.num_cores, _INFO.num_subcores
NW = NC * NS

NFULL = V // 128
TAIL = V - NFULL * 128
BASE_BLKS = NFULL // NW
EXTRA = NFULL - BASE_BLKS * NW


WO = 128
NBUF = 4


def _format_table(weight_t):
    mesh = plsc.VectorSubcoreMesh(core_axis_name="c", subcore_axis_name="s")

    @functools.partial(
        pl.kernel,
        mesh=mesh,
        out_type=jax.ShapeDtypeStruct((V // 4, 128), jnp.float32),
        compiler_params=pltpu.CompilerParams(
            needs_layout_passes=False, use_tc_tiling_on_sc=True
        ),
        scratch_types=[pltpu.VMEM((NBUF, D, 128), jnp.float32)]
        + [pltpu.VMEM((NBUF, D, WO), jnp.float32)]
        + [pltpu.SemaphoreType.DMA] * NBUF
        + [pltpu.SemaphoreType.DMA] * NBUF,
    )
    def k(wt_hbm, out_hbm, s_v, o_v, *sems):
        sin = sems[:NBUF]
        sout = sems[NBUF:]
        wid = lax.axis_index("s") * NC + lax.axis_index("c")
        nblk = jnp.where(wid < EXTRA, BASE_BLKS + 1, BASE_BLKS)
        b0 = wid * BASE_BLKS + jnp.minimum(wid, EXTRA)

        lanes = lax.iota(jnp.int32, L)
        rows_g = [(lanes + 16 * g) // 4 for g in range(8)]
        cols_g = [((lanes + 16 * g) % 4) * 32 for g in range(8)]

        def start_in(g, buf):
            c0 = pl.multiple_of((b0 + g) * 128, 128)
            pltpu.async_copy(
                wt_hbm.at[:, pl.ds(c0, 128)], s_v.at[buf], sin[buf]
            )

        def wait_in(g, buf):
            c0 = pl.multiple_of((b0 + g) * 128, 128)
            pltpu.make_async_copy(
                wt_hbm.at[:, pl.ds(c0, 128)], s_v.at[buf], sin[buf]
            ).wait()

        def start_out(g, buf):
            p0 = pl.multiple_of((b0 + g) * 32, 32)
            pltpu.async_copy(
                o_v.at[buf], out_hbm.at[pl.ds(p0, 32)], sout[buf]
            )

        def wait_out(g, buf):
            p0 = pl.multiple_of((b0 + g) * 32, 32)
            pltpu.make_async_copy(
                o_v.at[buf], out_hbm.at[pl.ds(p0, 32)], sout[buf]
            ).wait()

        def transpose_block(buf, ncols):
            for d in range(D):
                vals = [
                    s_v[buf, d, pl.ds(16 * g, L)] for g in range(ncols // 16)
                ]
                for g in range(ncols // 16):
                    plsc.store_scatter(
                        o_v.at[buf], [rows_g[g], cols_g[g] + d], vals[g]
                    )

        for buf in range(NBUF):
            @pl.when(buf < nblk)
            def _(buf=buf):
                start_in(buf, buf)

        def ring_body(i, _):
            for buf in range(NBUF):
                g = i * NBUF + buf

                @pl.when(g < nblk)
                def _(g=g, buf=buf):
                    wait_in(g, buf)

                    @pl.when(g >= NBUF)
                    def _():
                        wait_out(g - NBUF, buf)

                    transpose_block(buf, 128)
                    start_out(g, buf)

                    @pl.when(g + NBUF < nblk)
                    def _():
                        start_in(g + NBUF, buf)

            return 0

        n_iters = (BASE_BLKS + 1 + NBUF - 1) // NBUF
        lax.fori_loop(0, n_iters, ring_body, 0, unroll=False)

        for buf in range(NBUF):
            wait_out(0, buf)

        @pl.when(wid == NW - 1)
        def _():
            for d in range(D):
                pltpu.sync_copy(
                    wt_hbm.at[d, pl.ds(NFULL * 128, TAIL)],
                    s_v.at[0, d, pl.ds(0, TAIL)],
                )
            for d in range(D):
                for g in range(TAIL // 16):
                    vals = s_v[0, d, pl.ds(16 * g, L)]
                    plsc.store_scatter(
                        o_v.at[0], [rows_g[g], cols_g[g] + d], vals
                    )
            pltpu.sync_copy(
                o_v.at[0, pl.ds(0, TAIL // 4)],
                out_hbm.at[pl.ds(NFULL * 32, TAIL // 4)],
            )

    return k(weight_t)


def _lane_min(v):
    dnums = lax.GatherDimensionNumbers(
        offset_dims=(), collapsed_slice_dims=(0,), start_index_map=(0,)
    )
    for sh in (8, 4, 2, 1):
        perm = (lax.iota(jnp.int32, L) + sh) % L
        rot = lax.gather(
            v, perm[:, None], dnums, (1,),
            mode=lax.GatherScatterMode.PROMISE_IN_BOUNDS,
        )
        v = jnp.minimum(v, rot)
    return v[0]


def _gather_rows(table, idx, B, C):
    b_per_w = B // NW
    n_chunks = b_per_w // C
    gathers_per_chunk = C // IB
    mesh = plsc.VectorSubcoreMesh(core_axis_name="c", subcore_axis_name="s")

    @functools.partial(
        pl.kernel,
        mesh=mesh,
        out_type=jax.ShapeDtypeStruct((B, D), jnp.float32),
        compiler_params=pltpu.CompilerParams(
            needs_layout_passes=False, use_tc_tiling_on_sc=False
        ),
        scratch_types=[
            pltpu.VMEM((C,), jnp.int32),
            pltpu.VMEM((C, D), jnp.float32),
            pltpu.SemaphoreType.DMA,
        ],
    )
    def k(table_hbm, idx_hbm, out_hbm, idx_v, rows_v, sem):
        wid = lax.axis_index("s") * NC + lax.axis_index("c")
        base = wid * b_per_w
        zeros = jnp.zeros((L,), jnp.float32)

        def chunk_body(g, _):
            off = pl.multiple_of(base + g * C, C)
            pltpu.sync_copy(idx_hbm.at[pl.ds(off, C)], idx_v)
            for j in range(gathers_per_chunk):
                pltpu.async_copy(
                    table_hbm.at[idx_v.at[pl.ds(j * IB, IB)]],
                    rows_v.at[pl.ds(j * IB, IB)],
                    sem,
                )

            def min_body(i, acc):
                return jnp.minimum(acc, idx_v[pl.ds(i * L, L)])

            acc = lax.fori_loop(
                0, C // L, min_body, jnp.full((L,), 2**30, jnp.int32),
                unroll=False,
            )
            min_idx = _lane_min(acc)

            for j in range(gathers_per_chunk):
                pltpu.make_async_copy(
                    table_hbm.at[idx_v.at[pl.ds(j * IB, IB)]],
                    rows_v.at[pl.ds(j * IB, IB)],
                    sem,
                ).wait()

            @pl.when(min_idx == PAD)
            def _():
                def fix_body(i, _):
                    iv = idx_v[pl.ds(i * L, L)]
                    m = iv == PAD
                    rowpos = lax.iota(jnp.int32, L) + i * L
                    for j in range(D):
                        plsc.store_scatter(
                            rows_v,
                            [rowpos, jnp.full((L,), j, jnp.int32)],
                            zeros,
                            mask=m,
                        )
                    return 0

                lax.fori_loop(0, C // L, fix_body, 0, unroll=False)

            pltpu.sync_copy(rows_v, out_hbm.at[pl.ds(off, C)])
            return 0

        lax.fori_loop(0, n_chunks, chunk_body, 0, unroll=False)

    return k(table, idx)


@functools.partial(jax.jit, static_argnums=(2,))
def _emb_lookup(weight, idx, B):
    table_packed = _format_table(jnp.swapaxes(weight, 0, 1))
    table = table_packed.reshape(V, D)
    return _gather_rows(table, idx, B, 1024)


def kernel(input_, weight):
    shape = input_.shape
    B = input_.size
    idx = input_.reshape(B)
    out = _emb_lookup(weight, idx, B)
    return out.reshape(*shape, D)
